# baseline (speedup 1.0000x reference)
import functools

import jax
import jax.numpy as jnp
from jax import lax
from jax.experimental import pallas as pl
from jax.experimental.pallas import tpu as pltpu
from jax.experimental.pallas import tpu_sc as plsc

N = 10000
E = 320000
G = 32
D = 128
CP = 16
BE = 2000

_f32 = jnp.float32


def _embed_body(a_ref, c_ref, b_ref, t_ref, atom_ref, charge_ref,
                wn_ref, bn_ref, wt_ref, bt_ref, h_ref):
    af = a_ref[...]
    cf = c_ref[...]
    bf = b_ref[...]
    aoh = (af == lax.broadcasted_iota(jnp.int32, (N, 16), 1).astype(_f32)).astype(_f32)
    coh = (cf == lax.broadcasted_iota(jnp.int32, (N, 8), 1).astype(_f32)).astype(_f32)
    boh = (bf == lax.broadcasted_iota(jnp.int32, (N, G), 1).astype(_f32)).astype(_f32)
    ones = jnp.ones((N, 1), _f32)
    counts = lax.dot_general(boh, ones, (((0,), (0,)), ((), ())),
                             preferred_element_type=_f32)
    n_tbl = jnp.log1p(counts) @ wn_ref[...] + bn_ref[...]
    t_tbl = t_ref[...] @ wt_ref[...] + bt_ref[...]
    h_ref[...] = jnp.concatenate(
        [aoh @ atom_ref[...], coh @ charge_ref[...],
         boh @ n_tbl, boh @ t_tbl], axis=1)


def _embed_call(a_f, c_f, b_f, t_col, p):
    return pl.pallas_call(
        _embed_body,
        out_shape=jax.ShapeDtypeStruct((N, D), _f32),
    )(a_f, c_f, b_f, t_col, p['atom_emb'], p['charge_emb'],
      p['Wn'], p['bn'].reshape(1, -1), p['Wt'], p['bt'].reshape(1, -1))


def _edge_body(hr_ref, hc_ref, d2_ref, e_ref,
               A_ref, B_ref, wc_ref, wd_ref, eemb_ref, be1_ref,
               W2_ref, be2_ref, wx_ref, bx_ref,
               m2_ref, w_ref):
    eoh = (e_ref[...] == lax.broadcasted_iota(jnp.int32, (BE, 5), 1).astype(_f32)).astype(_f32)
    te = eemb_ref[...] @ wd_ref[...]
    z1 = (jnp.dot(hr_ref[...], A_ref[...], preferred_element_type=_f32)
          + jnp.dot(hc_ref[...], B_ref[...], preferred_element_type=_f32)
          + d2_ref[...] * wc_ref[...] + eoh @ te + be1_ref[...])
    m1 = z1 * jax.nn.sigmoid(z1)
    z2 = jnp.dot(m1, W2_ref[...], preferred_element_type=_f32) + be2_ref[...]
    m2 = z2 * jax.nn.sigmoid(z2)
    m2_ref[...] = m2
    w_ref[...] = jnp.sum(m2 * wx_ref[...], axis=1, keepdims=True) + bx_ref[...]


def _edge_call(hr, hc, d2e, e_f, lp, edge_emb):
    nb = E // BE
    A = lp['We1'][0:D]
    B = lp['We1'][D:2 * D]
    wc = lp['We1'][2 * D:2 * D + 1]
    Wd = lp['We1'][2 * D + 1:]
    eb = lambda w: pl.BlockSpec((BE, w), lambda i: (i, 0))
    fb = lambda arr: pl.BlockSpec(arr.shape, lambda i: (0,) * arr.ndim)
    args = (hr, hc, d2e, e_f, A, B, wc, Wd, edge_emb,
            lp['be1'].reshape(1, -1), lp['We2'], lp['be2'].reshape(1, -1),
            lp['Wx'].reshape(1, -1), lp['bx'].reshape(1, -1))
    specs = [eb(D), eb(D), eb(1), eb(1)] + [fb(a) for a in args[4:]]
    return pl.pallas_call(
        _edge_body,
        grid=(nb,),
        in_specs=specs,
        out_specs=[eb(D), eb(1)],
        out_shape=[jax.ShapeDtypeStruct((E, D), _f32),
                   jax.ShapeDtypeStruct((E, 1), _f32)],
    )(*args)


def _node_body(h_ref, cp_ref, a0_ref, a1_ref, c0_ref, c1_ref,
               wa_ref, wb_ref, bh1_ref, w2_ref, bh2_ref,
               hout_ref, cpout_ref):
    agg = a0_ref[...] + a1_ref[...]
    cd = c0_ref[...] + c1_ref[...]
    deg = cd[:, 3:4] + 1.0
    cpout_ref[...] = cp_ref[...] + jnp.concatenate(
        [cd[:, 0:3] / deg, jnp.zeros((N, CP - 3), _f32)], axis=1)
    z = (jnp.dot(h_ref[...], wa_ref[...], preferred_element_type=_f32)
         + jnp.dot(agg, wb_ref[...], preferred_element_type=_f32)
         + bh1_ref[...])
    hu = z * jax.nn.sigmoid(z)
    hout_ref[...] = (h_ref[...] +
                     jnp.dot(hu, w2_ref[...], preferred_element_type=_f32)
                     + bh2_ref[...])


def _node_call(h, coordp, agg0, agg1, cd0, cd1, lp):
    return pl.pallas_call(
        _node_body,
        out_shape=[jax.ShapeDtypeStruct((N, D), _f32),
                   jax.ShapeDtypeStruct((N, CP), _f32)],
    )(h, coordp, agg0, agg1, cd0, cd1,
      lp['Wh1'][0:D], lp['Wh1'][D:], lp['bh1'].reshape(1, -1),
      lp['Wh2'], lp['bh2'].reshape(1, -1))


def _heads_body(h_ref, cp_ref, wa_ref, ba_ref, wc_ref, bc_ref,
                wm_ref, bm_ref, ww_ref, bw_ref,
                al_ref, cl_ref, co_ref, mm_ref, lw_ref):
    h = h_ref[...]
    al_ref[...] = jnp.dot(h, wa_ref[...], preferred_element_type=_f32) + ba_ref[...]
    cl_ref[...] = jnp.dot(h, wc_ref[...], preferred_element_type=_f32) + bc_ref[...]
    ct = cp_ref[:, 0:3]
    co_ref[...] = ct
    mm = jnp.dot(h, wm_ref[...], preferred_element_type=_f32) + bm_ref[...]
    mm_ref[...] = mm + jnp.concatenate([ct, ct, ct, ct], axis=1)
    zw = jnp.dot(h, ww_ref[...], preferred_element_type=_f32) + bw_ref[...]
    zmax = jnp.max(zw, axis=1, keepdims=True)
    s = zw - zmax
    lw_ref[...] = s - jnp.log(jnp.sum(jnp.exp(s), axis=1, keepdims=True))


def _heads_call(h, coordp, p):
    return pl.pallas_call(
        _heads_body,
        out_shape=[jax.ShapeDtypeStruct((N, 16), _f32),
                   jax.ShapeDtypeStruct((N, 8), _f32),
                   jax.ShapeDtypeStruct((N, 3), _f32),
                   jax.ShapeDtypeStruct((N, 12), _f32),
                   jax.ShapeDtypeStruct((N, 4), _f32)],
    )(h, coordp, p['Wa'], p['ba'].reshape(1, -1), p['Wc'], p['bc'].reshape(1, -1),
      p['Wm'], p['bm'].reshape(1, -1), p['Ww'], p['bw'].reshape(1, -1))


_NC, _NS = 2, 16
_NW = _NC * _NS
_EPW = E // _NW
_GK = 80
_GCH = _EPW // _GK


def _gather_body(h_hbm, cpf_hbm, row_hbm, col_hbm,
                 hr_hbm, hc_hbm, d2_hbm,
                 cp_v, idxr_v, idxc_v, hr_v, hc_v, d2_v, sem):
    wid = lax.axis_index("s") * _NC + lax.axis_index("c")
    wbase = wid * _EPW
    pltpu.sync_copy(cpf_hbm, cp_v)

    def chunk(j, carry):
        base = wbase + j * _GK
        pltpu.sync_copy(row_hbm.at[pl.ds(base, _GK)], idxr_v)
        pltpu.sync_copy(col_hbm.at[pl.ds(base, _GK)], idxc_v)
        g1 = pltpu.async_copy(h_hbm.at[idxr_v], hr_v, sem)
        g2 = pltpu.async_copy(h_hbm.at[idxc_v], hc_v, sem)
        for k in range(_GK // 16):
            sl = pl.ds(k * 16, 16)
            rb = idxr_v[sl] * 4
            cb = idxc_v[sl] * 4
            dx = plsc.load_gather(cp_v, [rb]) - plsc.load_gather(cp_v, [cb])
            dy = plsc.load_gather(cp_v, [rb + 1]) - plsc.load_gather(cp_v, [cb + 1])
            dz = plsc.load_gather(cp_v, [rb + 2]) - plsc.load_gather(cp_v, [cb + 2])
            d2_v[sl] = dx * dx + dy * dy + dz * dz
        g1.wait(); g2.wait()
        pltpu.sync_copy(hr_v, hr_hbm.at[pl.ds(base, _GK)])
        pltpu.sync_copy(hc_v, hc_hbm.at[pl.ds(base, _GK)])
        pltpu.sync_copy(d2_v, d2_hbm.at[pl.ds(base, _GK)])
        return carry

    lax.fori_loop(0, _GCH, chunk, 0)


def _gather_stage(h, cp_flat, row, col):
    f = pl.kernel(
        _gather_body,
        out_type=[jax.ShapeDtypeStruct((E, D), _f32),
                  jax.ShapeDtypeStruct((E, D), _f32),
                  jax.ShapeDtypeStruct((E,), _f32)],
        mesh=plsc.VectorSubcoreMesh(core_axis_name="c", subcore_axis_name="s"),
        compiler_params=pltpu.CompilerParams(needs_layout_passes=False),
        scratch_types=[pltpu.VMEM((4 * N,), _f32),
                       pltpu.VMEM((_GK,), jnp.int32),
                       pltpu.VMEM((_GK,), jnp.int32),
                       pltpu.VMEM((_GK, D), _f32),
                       pltpu.VMEM((_GK, D), _f32),
                       pltpu.VMEM((_GK,), _f32),
                       pltpu.SemaphoreType.DMA],
    )
    return f(h, cp_flat, row, col)


def _scatter_stage(m2, w, cp_flat, row, col):
    c4 = cp_flat.reshape(N, 4)
    diff = c4[row, 0:3] - c4[col, 0:3]
    wd4 = jnp.concatenate([diff * w, jnp.ones((E, 1), _f32)], axis=1)
    agg = jax.ops.segment_sum(m2, row, num_segments=N)
    cd = jax.ops.segment_sum(wd4, row, num_segments=N)
    return agg, jnp.zeros_like(agg), cd, jnp.zeros_like(cd)


def kernel(x, a, c, e, edge_index, batch, t, params):
    p = params
    row = edge_index[0].astype(jnp.int32)
    col = edge_index[1].astype(jnp.int32)
    a_f = a.astype(_f32).reshape(N, 1)
    c_f = c.astype(_f32).reshape(N, 1)
    b_f = batch.astype(_f32).reshape(N, 1)
    e_f = e.astype(_f32).reshape(E, 1)
    t_col = t.reshape(G, 1)
    coordp = jnp.pad(x, ((0, 0), (0, CP - 3)))

    h = _embed_call(a_f, c_f, b_f, t_col, p)
    for l in range(2):
        lp = p['layers'][l]
        cp_flat = coordp[:, 0:4].reshape(4 * N)
        hr, hc, d2e = _gather_stage(h, cp_flat, row, col)
        m2, w = _edge_call(hr, hc, d2e.reshape(E, 1), e_f, lp, p['edge_emb'])
        agg0, agg1, cd0, cd1 = _scatter_stage(m2, w, cp_flat, row, col)
        h, coordp = _node_call(h, coordp, agg0, agg1, cd0, cd1, lp)

    al, cl, co, mm, lw = _heads_call(h, coordp, p)
    return al, cl, co, mm.reshape(N, 4, 3), lw, h


# speedup vs baseline: 19.1569x; 19.1569x over previous
import functools

import jax
import jax.numpy as jnp
from jax import lax
from jax.experimental import pallas as pl
from jax.experimental.pallas import tpu as pltpu
from jax.experimental.pallas import tpu_sc as plsc

N = 10000
E = 320000
G = 32
D = 128
CP = 16
BE = 8000

_f32 = jnp.float32


def _embed_body(a_ref, c_ref, b_ref, t_ref, atom_ref, charge_ref,
                wn_ref, bn_ref, wt_ref, bt_ref, h_ref):
    af = a_ref[...]
    cf = c_ref[...]
    bf = b_ref[...]
    aoh = (af == lax.broadcasted_iota(jnp.int32, (N, 16), 1).astype(_f32)).astype(_f32)
    coh = (cf == lax.broadcasted_iota(jnp.int32, (N, 8), 1).astype(_f32)).astype(_f32)
    boh = (bf == lax.broadcasted_iota(jnp.int32, (N, G), 1).astype(_f32)).astype(_f32)
    ones = jnp.ones((N, 1), _f32)
    counts = lax.dot_general(boh, ones, (((0,), (0,)), ((), ())),
                             preferred_element_type=_f32)
    n_tbl = jnp.log1p(counts) @ wn_ref[...] + bn_ref[...]
    t_tbl = t_ref[...] @ wt_ref[...] + bt_ref[...]
    h_ref[...] = jnp.concatenate(
        [aoh @ atom_ref[...], coh @ charge_ref[...],
         boh @ n_tbl, boh @ t_tbl], axis=1)


def _embed_call(a_f, c_f, b_f, t_col, p):
    return pl.pallas_call(
        _embed_body,
        out_shape=jax.ShapeDtypeStruct((N, D), _f32),
    )(a_f, c_f, b_f, t_col, p['atom_emb'], p['charge_emb'],
      p['Wn'], p['bn'].reshape(1, -1), p['Wt'], p['bt'].reshape(1, -1))


def _edge_body(hr_ref, hc_ref, d2_ref, e_ref,
               A_ref, B_ref, wc_ref, wd_ref, eemb_ref, be1_ref,
               W2_ref, be2_ref, wx_ref, bx_ref,
               m2_ref, w_ref):
    eoh = (e_ref[...] == lax.broadcasted_iota(jnp.int32, (BE, 5), 1).astype(_f32)).astype(_f32)
    te = eemb_ref[...] @ wd_ref[...]
    z1 = (jnp.dot(hr_ref[...], A_ref[...], preferred_element_type=_f32)
          + jnp.dot(hc_ref[...], B_ref[...], preferred_element_type=_f32)
          + d2_ref[...] * wc_ref[...] + eoh @ te + be1_ref[...])
    m1 = z1 * jax.nn.sigmoid(z1)
    z2 = jnp.dot(m1, W2_ref[...], preferred_element_type=_f32) + be2_ref[...]
    m2 = z2 * jax.nn.sigmoid(z2)
    m2_ref[...] = m2
    w_ref[...] = jnp.sum(m2 * wx_ref[...], axis=1, keepdims=True) + bx_ref[...]


def _edge_call(hr, hc, d2e, e_f, lp, edge_emb):
    nb = E // BE
    A = lp['We1'][0:D]
    B = lp['We1'][D:2 * D]
    wc = lp['We1'][2 * D:2 * D + 1]
    Wd = lp['We1'][2 * D + 1:]
    eb = lambda w: pl.BlockSpec((BE, w), lambda i: (i, 0))
    fb = lambda arr: pl.BlockSpec(arr.shape, lambda i: (0,) * arr.ndim)
    args = (hr, hc, d2e, e_f, A, B, wc, Wd, edge_emb,
            lp['be1'].reshape(1, -1), lp['We2'], lp['be2'].reshape(1, -1),
            lp['Wx'].reshape(1, -1), lp['bx'].reshape(1, -1))
    specs = [eb(D), eb(D), eb(1), eb(1)] + [fb(a) for a in args[4:]]
    return pl.pallas_call(
        _edge_body,
        grid=(nb,),
        in_specs=specs,
        out_specs=[eb(D), eb(1)],
        out_shape=[jax.ShapeDtypeStruct((E, D), _f32),
                   jax.ShapeDtypeStruct((E, 1), _f32)],
    )(*args)


def _node_body(h_ref, cp_ref, a0_ref, a1_ref, c0_ref, c1_ref,
               wa_ref, wb_ref, bh1_ref, w2_ref, bh2_ref,
               hout_ref, cpout_ref):
    agg = a0_ref[...] + a1_ref[...]
    cd = c0_ref[...] + c1_ref[...]
    deg = cd[:, 3:4] + 1.0
    cpout_ref[...] = cp_ref[...] + jnp.concatenate(
        [cd[:, 0:3] / deg, jnp.zeros((N, CP - 3), _f32)], axis=1)
    z = (jnp.dot(h_ref[...], wa_ref[...], preferred_element_type=_f32)
         + jnp.dot(agg, wb_ref[...], preferred_element_type=_f32)
         + bh1_ref[...])
    hu = z * jax.nn.sigmoid(z)
    hout_ref[...] = (h_ref[...] +
                     jnp.dot(hu, w2_ref[...], preferred_element_type=_f32)
                     + bh2_ref[...])


def _node_call(h, coordp, agg0, agg1, cd0, cd1, lp):
    return pl.pallas_call(
        _node_body,
        out_shape=[jax.ShapeDtypeStruct((N, D), _f32),
                   jax.ShapeDtypeStruct((N, CP), _f32)],
    )(h, coordp, agg0, agg1, cd0, cd1,
      lp['Wh1'][0:D], lp['Wh1'][D:], lp['bh1'].reshape(1, -1),
      lp['Wh2'], lp['bh2'].reshape(1, -1))


def _heads_body(h_ref, cp_ref, wa_ref, ba_ref, wc_ref, bc_ref,
                wm_ref, bm_ref, ww_ref, bw_ref,
                al_ref, cl_ref, co_ref, mm_ref, lw_ref):
    h = h_ref[...]
    al_ref[...] = jnp.dot(h, wa_ref[...], preferred_element_type=_f32) + ba_ref[...]
    cl_ref[...] = jnp.dot(h, wc_ref[...], preferred_element_type=_f32) + bc_ref[...]
    ct = cp_ref[:, 0:3]
    co_ref[...] = ct
    mm = jnp.dot(h, wm_ref[...], preferred_element_type=_f32) + bm_ref[...]
    mm_ref[...] = mm + jnp.concatenate([ct, ct, ct, ct], axis=1)
    zw = jnp.dot(h, ww_ref[...], preferred_element_type=_f32) + bw_ref[...]
    zmax = jnp.max(zw, axis=1, keepdims=True)
    s = zw - zmax
    lw_ref[...] = s - jnp.log(jnp.sum(jnp.exp(s), axis=1, keepdims=True))


def _heads_call(h, coordp, p):
    return pl.pallas_call(
        _heads_body,
        out_shape=[jax.ShapeDtypeStruct((N, 16), _f32),
                   jax.ShapeDtypeStruct((N, 8), _f32),
                   jax.ShapeDtypeStruct((N, 3), _f32),
                   jax.ShapeDtypeStruct((N, 12), _f32),
                   jax.ShapeDtypeStruct((N, 4), _f32)],
    )(h, coordp, p['Wa'], p['ba'].reshape(1, -1), p['Wc'], p['bc'].reshape(1, -1),
      p['Wm'], p['bm'].reshape(1, -1), p['Ww'], p['bw'].reshape(1, -1))


_NC, _NS = 2, 16
_NW = _NC * _NS
_EPW = E // _NW
_GK = 80
_GCH = _EPW // _GK


def _gather_body(h_hbm, cpf_hbm, row_hbm, col_hbm,
                 hr_hbm, hc_hbm, dx_hbm, dy_hbm, dz_hbm, d2_hbm,
                 cp_v, idxr_all, idxc_all,
                 hr0, hc0, dx0, dy0, dz0, d20,
                 hr1, hc1, dx1, dy1, dz1, d21, sem, semw):
    wid = lax.axis_index("s") * _NC + lax.axis_index("c")
    wbase = wid * _EPW
    pltpu.sync_copy(cpf_hbm, cp_v)
    pltpu.sync_copy(row_hbm.at[pl.ds(wbase, _EPW)], idxr_all)
    pltpu.sync_copy(col_hbm.at[pl.ds(wbase, _EPW)], idxc_all)
    set0 = (hr0, hc0, dx0, dy0, dz0, d20)
    set1 = (hr1, hc1, dx1, dy1, dz1, d21)

    def start(c, bufs):
        hr_v, hc_v = bufs[0], bufs[1]
        base = wbase + c * _GK
        loc = c * _GK
        g1 = pltpu.async_copy(h_hbm.at[idxr_all.at[pl.ds(loc, _GK)]], hr_v, sem)
        g2 = pltpu.async_copy(h_hbm.at[idxc_all.at[pl.ds(loc, _GK)]], hc_v, sem)
        return base, g1, g2

    def finish(base, g1, g2, bufs):
        hr_v, hc_v, dx_v, dy_v, dz_v, d2_v = bufs
        loc = base - wbase
        for k in range(_GK // 16):
            sl = pl.ds(k * 16, 16)
            rb = idxr_all[pl.dslice(loc + k * 16, 16)] * 4
            cb = idxc_all[pl.dslice(loc + k * 16, 16)] * 4
            sl = pl.ds(k * 16, 16)
            dx = plsc.load_gather(cp_v, [rb]) - plsc.load_gather(cp_v, [cb])
            dy = plsc.load_gather(cp_v, [rb + 1]) - plsc.load_gather(cp_v, [cb + 1])
            dz = plsc.load_gather(cp_v, [rb + 2]) - plsc.load_gather(cp_v, [cb + 2])
            dx_v[sl] = dx
            dy_v[sl] = dy
            dz_v[sl] = dz
            d2_v[sl] = dx * dx + dy * dy + dz * dz
        g1.wait(); g2.wait()
        sl = pl.ds(base, _GK)
        return (pltpu.async_copy(hr_v, hr_hbm.at[sl], semw),
                pltpu.async_copy(hc_v, hc_hbm.at[sl], semw),
                pltpu.async_copy(dx_v, dx_hbm.at[sl], semw),
                pltpu.async_copy(dy_v, dy_hbm.at[sl], semw),
                pltpu.async_copy(dz_v, dz_hbm.at[sl], semw),
                pltpu.async_copy(d2_v, d2_hbm.at[sl], semw))

    def chunkpair(j, carry):
        b0, g01, g02 = start(2 * j, set0)
        b1, g11, g12 = start(2 * j + 1, set1)
        w0 = finish(b0, g01, g02, set0)
        w1 = finish(b1, g11, g12, set1)
        for w in w0 + w1:
            w.wait()
        return carry

    lax.fori_loop(0, _GCH // 2, chunkpair, 0)
    b, g1, g2 = start(_GCH - 1, set0)
    for w in finish(b, g1, g2, set0):
        w.wait()


def _gather_stage(h, cp_flat, row, col):
    f = pl.kernel(
        _gather_body,
        out_type=[jax.ShapeDtypeStruct((E, D), _f32),
                  jax.ShapeDtypeStruct((E, D), _f32),
                  jax.ShapeDtypeStruct((E,), _f32),
                  jax.ShapeDtypeStruct((E,), _f32),
                  jax.ShapeDtypeStruct((E,), _f32),
                  jax.ShapeDtypeStruct((E,), _f32)],
        mesh=plsc.VectorSubcoreMesh(core_axis_name="c", subcore_axis_name="s"),
        compiler_params=pltpu.CompilerParams(needs_layout_passes=False, skip_device_barrier=True),
        scratch_types=[pltpu.VMEM((4 * N,), _f32),
                       pltpu.VMEM((_EPW,), jnp.int32),
                       pltpu.VMEM((_EPW,), jnp.int32)] +
                      [pltpu.VMEM((_GK, D), _f32),
                       pltpu.VMEM((_GK, D), _f32),
                       pltpu.VMEM((_GK,), _f32),
                       pltpu.VMEM((_GK,), _f32),
                       pltpu.VMEM((_GK,), _f32),
                       pltpu.VMEM((_GK,), _f32)] * 2 +
                      [pltpu.SemaphoreType.DMA,
                       pltpu.SemaphoreType.DMA],
    )
    return f(h, cp_flat, row, col)


_RB = 632
_RL = N - 15 * _RB


def _scatter_body(m2_hbm, w_hbm, dx_hbm, dy_hbm, dz_hbm, row_hbm,
                  agg0_hbm, agg1_hbm, cd0_hbm, cd1_hbm,
                  idxr_v, m2_v, w_v,
                  dxw_v, dyw_v, dzw_v, i0_v, i1_v, i2_v, i3_v,
                  idxr_b, m2_b, w_b,
                  dxw_b, dyw_b, dzw_b, i0_b, i1_b, i2_b, i3_b,
                  ones_v, zf_v, agg_s, cd_s, sem, semw):
    cid = lax.axis_index("c")
    sid = lax.axis_index("s")
    wid = sid * _NC + cid
    wbase = wid * _EPW

    z16 = jnp.zeros((16,), _f32)
    for r in range(_GK):
        for k in range(D // 16):
            m2_v[r, pl.ds(k * 16, 16)] = z16
    for k in range(_RB * 4 // 16):
        zf_v[pl.ds(k * 16, 16)] = z16
    start = sid * _RB

    @pl.when(sid < 15)
    def _():
        for i in range(7):
            pltpu.sync_copy(m2_v, agg_s.at[pl.ds(start + i * _GK, _GK)])
        pltpu.sync_copy(m2_v.at[pl.ds(0, _RB - 7 * _GK)],
                        agg_s.at[pl.ds(start + 7 * _GK, _RB - 7 * _GK)])
        pltpu.sync_copy(zf_v, cd_s.at[pl.ds(start * 4, _RB * 4)])

    @pl.when(sid == 15)
    def _():
        for i in range(6):
            pltpu.sync_copy(m2_v, agg_s.at[pl.ds(15 * _RB + i * _GK, _GK)])
        pltpu.sync_copy(m2_v.at[pl.ds(0, _RL - 6 * _GK)],
                        agg_s.at[pl.ds(15 * _RB + 6 * _GK, _RL - 6 * _GK)])
        pltpu.sync_copy(zf_v.at[pl.ds(0, _RL * 4)],
                        cd_s.at[pl.ds(15 * _RB * 4, _RL * 4)])

    for k in range(_GK // 16):
        ones_v[pl.ds(k * 16, 16)] = jnp.full((16,), 1.0, _f32)
    plsc.subcore_barrier()

    sc_set0 = (idxr_v, m2_v, w_v, dxw_v, dyw_v, dzw_v, i0_v, i1_v, i2_v, i3_v)
    sc_set1 = (idxr_b, m2_b, w_b, dxw_b, dyw_b, dzw_b, i0_b, i1_b, i2_b, i3_b)

    def sload(c, bufs):
        idxr, m2b, wb, dxb, dyb, dzb = bufs[0], bufs[1], bufs[2], bufs[3], bufs[4], bufs[5]
        base = wbase + c * _GK
        pltpu.sync_copy(row_hbm.at[pl.ds(base, _GK)], idxr)
        g1 = pltpu.async_copy(m2_hbm.at[pl.ds(base, _GK)], m2b, sem)
        pltpu.sync_copy(w_hbm.at[pl.ds(base, _GK)], wb)
        pltpu.sync_copy(dx_hbm.at[pl.ds(base, _GK)], dxb)
        pltpu.sync_copy(dy_hbm.at[pl.ds(base, _GK)], dyb)
        pltpu.sync_copy(dz_hbm.at[pl.ds(base, _GK)], dzb)
        return g1

    def sfin(g1, bufs):
        idxr, m2b, wb, dxb, dyb, dzb, i0, i1, i2, i3 = bufs
        for k in range(_GK // 16):
            sl = pl.ds(k * 16, 16)
            rb = idxr[sl] * 4
            w16 = wb[sl]
            dxb[sl] = dxb[sl] * w16
            dyb[sl] = dyb[sl] * w16
            dzb[sl] = dzb[sl] * w16
            i0[sl] = rb
            i1[sl] = rb + 1
            i2[sl] = rb + 2
            i3[sl] = rb + 3
        g1.wait()
        return (pltpu.async_copy(m2b, agg_s.at[idxr], semw, add=True),
                pltpu.async_copy(dxb, cd_s.at[i0], semw, add=True),
                pltpu.async_copy(dyb, cd_s.at[i1], semw, add=True),
                pltpu.async_copy(dzb, cd_s.at[i2], semw, add=True),
                pltpu.async_copy(ones_v, cd_s.at[i3], semw, add=True))

    def chunkpair(j, carry):
        g0 = sload(2 * j, sc_set0)
        g1 = sload(2 * j + 1, sc_set1)
        a0 = sfin(g0, sc_set0)
        a1 = sfin(g1, sc_set1)
        for a in a0 + a1:
            a.wait()
        return carry

    lax.fori_loop(0, _GCH // 2, chunkpair, 0)
    g = sload(_GCH - 1, sc_set0)
    for a in sfin(g, sc_set0):
        a.wait()
    plsc.subcore_barrier()

    def emit(agg_hbm, cd_hbm):
        @pl.when(sid < 15)
        def _():
            for i in range(7):
                pltpu.sync_copy(agg_s.at[pl.ds(start + i * _GK, _GK)], m2_v)
                pltpu.sync_copy(m2_v, agg_hbm.at[pl.ds(start + i * _GK, _GK)])
            rem = _RB - 7 * _GK
            pltpu.sync_copy(agg_s.at[pl.ds(start + 7 * _GK, rem)],
                            m2_v.at[pl.ds(0, rem)])
            pltpu.sync_copy(m2_v.at[pl.ds(0, rem)],
                            agg_hbm.at[pl.ds(start + 7 * _GK, rem)])
            pltpu.sync_copy(cd_s.at[pl.ds(start * 4, _RB * 4)], zf_v)
            pltpu.sync_copy(zf_v, cd_hbm.at[pl.ds(start * 4, _RB * 4)])

        @pl.when(sid == 15)
        def _():
            for i in range(6):
                pltpu.sync_copy(agg_s.at[pl.ds(15 * _RB + i * _GK, _GK)], m2_v)
                pltpu.sync_copy(m2_v, agg_hbm.at[pl.ds(15 * _RB + i * _GK, _GK)])
            rem = _RL - 6 * _GK
            pltpu.sync_copy(agg_s.at[pl.ds(15 * _RB + 6 * _GK, rem)],
                            m2_v.at[pl.ds(0, rem)])
            pltpu.sync_copy(m2_v.at[pl.ds(0, rem)],
                            agg_hbm.at[pl.ds(15 * _RB + 6 * _GK, rem)])
            pltpu.sync_copy(cd_s.at[pl.ds(15 * _RB * 4, _RL * 4)],
                            zf_v.at[pl.ds(0, _RL * 4)])
            pltpu.sync_copy(zf_v.at[pl.ds(0, _RL * 4)],
                            cd_hbm.at[pl.ds(15 * _RB * 4, _RL * 4)])

    @pl.when(cid == 0)
    def _():
        emit(agg0_hbm, cd0_hbm)

    @pl.when(cid == 1)
    def _():
        emit(agg1_hbm, cd1_hbm)


def _scatter_stage(m2, w, dxe, dye, dze, row):
    f = pl.kernel(
        _scatter_body,
        out_type=[jax.ShapeDtypeStruct((N, D), _f32),
                  jax.ShapeDtypeStruct((N, D), _f32),
                  jax.ShapeDtypeStruct((4 * N,), _f32),
                  jax.ShapeDtypeStruct((4 * N,), _f32)],
        mesh=plsc.VectorSubcoreMesh(core_axis_name="c", subcore_axis_name="s"),
        compiler_params=pltpu.CompilerParams(needs_layout_passes=False, skip_device_barrier=True),
        scratch_types=[pltpu.VMEM((_GK,), jnp.int32),
                       pltpu.VMEM((_GK, D), _f32),
                       pltpu.VMEM((_GK,), _f32),
                       pltpu.VMEM((_GK,), _f32),
                       pltpu.VMEM((_GK,), _f32),
                       pltpu.VMEM((_GK,), _f32),
                       pltpu.VMEM((_GK,), jnp.int32),
                       pltpu.VMEM((_GK,), jnp.int32),
                       pltpu.VMEM((_GK,), jnp.int32),
                       pltpu.VMEM((_GK,), jnp.int32)] * 2 +
                      [pltpu.VMEM((_GK,), _f32),
                       pltpu.VMEM((_RB * 4,), _f32),
                       pltpu.VMEM_SHARED((N, D), _f32),
                       pltpu.VMEM_SHARED((4 * N,), _f32),
                       pltpu.SemaphoreType.DMA,
                       pltpu.SemaphoreType.DMA],
    )
    agg0, agg1, cd0f, cd1f = f(m2, w, dxe, dye, dze, row)
    return agg0, agg1, cd0f.reshape(N, 4), cd1f.reshape(N, 4)


def kernel(x, a, c, e, edge_index, batch, t, params):
    p = params
    row = edge_index[0].astype(jnp.int32)
    col = edge_index[1].astype(jnp.int32)
    a_f = a.astype(_f32).reshape(N, 1)
    c_f = c.astype(_f32).reshape(N, 1)
    b_f = batch.astype(_f32).reshape(N, 1)
    e_f = e.astype(_f32).reshape(E, 1)
    t_col = t.reshape(G, 1)
    coordp = jnp.pad(x, ((0, 0), (0, CP - 3)))

    h = _embed_call(a_f, c_f, b_f, t_col, p)
    for l in range(2):
        lp = p['layers'][l]
        cp_flat = coordp[:, 0:4].reshape(4 * N)
        hr, hc, dxe, dye, dze, d2e = _gather_stage(h, cp_flat, row, col)
        m2, w = _edge_call(hr, hc, d2e.reshape(E, 1), e_f, lp, p['edge_emb'])
        agg0, agg1, cd0, cd1 = _scatter_stage(m2, w.reshape(E), dxe, dye, dze, row)
        h, coordp = _node_call(h, coordp, agg0, agg1, cd0, cd1, lp)

    al, cl, co, mm, lw = _heads_call(h, coordp, p)
    return al, cl, co, mm.reshape(N, 4, 3), lw, h
